# Initial kernel scaffold; baseline (speedup 1.0000x reference)
import jax
import jax.numpy as jnp
from jax.experimental import pallas as pl


def kernel(x, batch, edge_index, edge_weight, W1, b1, W2, b2, W3, b3):
    raise NotImplementedError("write your pallas kernel here")


# speedup vs baseline: 1.0008x; 1.0008x over previous
import jax
import jax.numpy as jnp
from jax.experimental import pallas as pl


def _pool_body(h_ref, batch_ref, o_ref):
    i = pl.program_id(0)
    n = pl.num_programs(0)

    @pl.when(i == 0)
    def _():
        o_ref[...] = jnp.zeros_like(o_ref)

    b = batch_ref[0, 0]
    h = h_ref[...]
    onehot = (b[None, :] == jax.lax.broadcasted_iota(jnp.int32, (32, 1), 0)).astype(
        jnp.float32
    )
    o_ref[...] += onehot @ h

    @pl.when(i == n - 1)
    def _():
        o_ref[...] = jax.nn.sigmoid(o_ref[...])


def _pool_sigmoid(h, batch):
    N = h.shape[0]
    BLK = 2000
    return pl.pallas_call(
        _pool_body,
        grid=(N // BLK,),
        in_specs=[
            pl.BlockSpec((BLK, 1), lambda i: (i, 0)),
            pl.BlockSpec((1, 1, BLK), lambda i: (i, 0, 0)),
        ],
        out_specs=pl.BlockSpec((32, 1), lambda i: (0, 0)),
        out_shape=jax.ShapeDtypeStruct((32, 1), jnp.float32),
    )(h, batch.reshape(N // BLK, 1, BLK))


def _gcn_norm(edge_index, edge_weight, num_nodes):
    row, col = edge_index[0], edge_index[1]
    deg = jax.ops.segment_sum(edge_weight, col, num_segments=num_nodes)
    deg_safe = jnp.where(deg > 0, deg, 1.0)
    dis = jnp.where(deg > 0, deg_safe**-0.5, 0.0)
    return dis[row] * edge_weight * dis[col]


def _tagconv(x, edge_index, norm, W, b):
    row, col = edge_index[0], edge_index[1]
    out = x @ W[0]
    h = x
    for k in range(1, W.shape[0]):
        h = jax.ops.segment_sum(norm[:, None] * h[row], col, num_segments=x.shape[0])
        out = out + h @ W[k]
    return out + b


def kernel(x, batch, edge_index, edge_weight, W1, b1, W2, b2, W3, b3):
    norm = _gcn_norm(edge_index, edge_weight, x.shape[0])
    h = _tagconv(x, edge_index, norm, W1, b1)
    h = jax.nn.relu(h)
    h = _tagconv(h, edge_index, norm, W2, b2)
    h = jax.nn.relu(h)
    h = _tagconv(h, edge_index, norm, W3, b3)
    return _pool_sigmoid(h, batch)


# speedup vs baseline: 6.3672x; 6.3624x over previous
import functools

import jax
import jax.numpy as jnp
from jax import lax
from jax.experimental import pallas as pl
from jax.experimental.pallas import tpu as pltpu
from jax.experimental.pallas import tpu_sc as plsc

N = 50000
E = 800000
G = 32
F = 64
H = 32

NP = 51200
EP = 819200
EPG = 128
GPC = 4
CHUNK = EPG * GPC
NSUB = 16
RPS = NP // NSUB
BN = 2048
NB = NP // BN

_MESH = plsc.VectorSubcoreMesh(core_axis_name="c", subcore_axis_name="s")
_f32 = jnp.float32
_i32 = jnp.int32


def _zero_slice(zsrc, acc, s):
    pltpu.sync_copy(zsrc, acc.at[pl.ds(s * RPS, RPS)])


def _sc_deg_body(col_hbm, w_hbm, z1_hbm, out_hbm, cbuf, wbuf, acc, sem):
    c = lax.axis_index("c")
    s = lax.axis_index("s")
    wid = c * NSUB + s
    _zero_slice(z1_hbm, acc, s)
    plsc.subcore_barrier()

    def chunk(ci, carry):
        gbase = wid * 200 + ci * GPC
        pltpu.sync_copy(col_hbm.at[pl.ds(gbase, GPC)], cbuf)
        pltpu.sync_copy(w_hbm.at[pl.ds(gbase * EPG, CHUNK)], wbuf)
        ds = [
            pltpu.async_copy(
                wbuf.at[pl.ds(g * EPG, EPG)], acc.at[cbuf.at[g]], sem, add=True
            )
            for g in range(GPC)
        ]
        for d in ds:
            d.wait()
        return carry

    lax.fori_loop(0, 200 // GPC, chunk, jnp.int32(0))
    plsc.subcore_barrier()

    @pl.when(c == 0)
    def _():
        pltpu.sync_copy(
            acc.at[pl.ds(s * RPS, RPS)], out_hbm.at[0, pl.ds(s * RPS, RPS)]
        )

    @pl.when(c == 1)
    def _():
        pltpu.sync_copy(
            acc.at[pl.ds(s * RPS, RPS)], out_hbm.at[1, pl.ds(s * RPS, RPS)]
        )


_sc_deg = functools.partial(
    pl.kernel,
    out_type=jax.ShapeDtypeStruct((2, NP), _f32),
    mesh=_MESH,
    compiler_params=pltpu.CompilerParams(needs_layout_passes=False, use_tc_tiling_on_sc=False),
    scratch_types=[
        pltpu.VMEM((GPC, EPG), _i32),
        pltpu.VMEM((CHUNK,), _f32),
        pltpu.VMEM_SHARED((NP,), _f32),
        pltpu.SemaphoreType.DMA,
    ],
)(_sc_deg_body)


def _hop32_half(m_hbm, t_hbm, s, row_hbm, col_hbm, w_hbm, z2_hbm, rbuf, cbuf,
                wbuf, gbuf, acc, sem):
    pltpu.sync_copy(z2_hbm, acc.at[pl.ds(s * RPS, RPS)])
    plsc.subcore_barrier()

    def chunk(ci, carry):
        gbase = s * 400 + ci * GPC
        pltpu.sync_copy(row_hbm.at[pl.ds(gbase, GPC)], rbuf)
        pltpu.sync_copy(col_hbm.at[pl.ds(gbase, GPC)], cbuf)
        pltpu.sync_copy(w_hbm.at[pl.ds(gbase * EPG, CHUNK)], wbuf)
        gds = [
            pltpu.async_copy(
                m_hbm.at[rbuf.at[g]], gbuf.at[pl.ds(g * EPG, EPG)], sem
            )
            for g in range(GPC)
        ]
        for d in gds:
            d.wait()

        def scale(i, cc):
            for jj in range(16):
                e = i * 16 + jj
                sp = plsc.load_gather(wbuf, [jnp.full((16,), e, _i32)])
                gbuf[e, pl.ds(0, 16)] = gbuf[e, pl.ds(0, 16)] * sp
                gbuf[e, pl.ds(16, 16)] = gbuf[e, pl.ds(16, 16)] * sp
            return cc

        lax.fori_loop(0, CHUNK // 16, scale, jnp.int32(0))
        sds = [
            pltpu.async_copy(
                gbuf.at[pl.ds(g * EPG, EPG)], acc.at[cbuf.at[g]], sem, add=True
            )
            for g in range(GPC)
        ]
        for d in sds:
            d.wait()
        return carry

    lax.fori_loop(0, 400 // GPC, chunk, jnp.int32(0))
    plsc.subcore_barrier()
    pltpu.sync_copy(acc.at[pl.ds(s * RPS, RPS)], t_hbm.at[pl.ds(s * RPS, RPS)])


def _sc_hop32_body(ma_hbm, mb_hbm, row_hbm, col_hbm, w_hbm, z2_hbm, ta_hbm,
                   tb_hbm, rbuf, cbuf, wbuf, gbuf, acc, sem):
    c = lax.axis_index("c")
    s = lax.axis_index("s")

    @pl.when(c == 0)
    def _():
        _hop32_half(ma_hbm, ta_hbm, s, row_hbm, col_hbm, w_hbm, z2_hbm, rbuf,
                    cbuf, wbuf, gbuf, acc, sem)

    @pl.when(c == 1)
    def _():
        _hop32_half(mb_hbm, tb_hbm, s, row_hbm, col_hbm, w_hbm, z2_hbm, rbuf,
                    cbuf, wbuf, gbuf, acc, sem)


_sc_hop32 = functools.partial(
    pl.kernel,
    out_type=(
        jax.ShapeDtypeStruct((NP, H), _f32),
        jax.ShapeDtypeStruct((NP, H), _f32),
    ),
    mesh=_MESH,
    compiler_params=pltpu.CompilerParams(needs_layout_passes=False, use_tc_tiling_on_sc=False),
    scratch_types=[
        pltpu.VMEM((GPC, EPG), _i32),
        pltpu.VMEM((GPC, EPG), _i32),
        pltpu.VMEM((CHUNK,), _f32),
        pltpu.VMEM((CHUNK, H), _f32),
        pltpu.VMEM_SHARED((NP, H), _f32),
        pltpu.SemaphoreType.DMA,
    ],
)(_sc_hop32_body)


def _sc_hop1_body(m_hbm, row_hbm, col_hbm, w_hbm, z1_hbm, out_hbm, rbuf, cbuf,
                  wbuf, ubuf, mloc, acc, sem):
    c = lax.axis_index("c")
    s = lax.axis_index("s")
    wid = c * NSUB + s
    pltpu.sync_copy(m_hbm, mloc)
    _zero_slice(z1_hbm, acc, s)
    plsc.subcore_barrier()

    def chunk(ci, carry):
        gbase = wid * 200 + ci * GPC
        pltpu.sync_copy(row_hbm.at[pl.ds(gbase * EPG, CHUNK)], rbuf)
        pltpu.sync_copy(col_hbm.at[pl.ds(gbase, GPC)], cbuf)
        pltpu.sync_copy(w_hbm.at[pl.ds(gbase * EPG, CHUNK)], wbuf)

        def scale(i, cc):
            rv = rbuf[pl.ds(i * 16, 16)]
            wv = wbuf[pl.ds(i * 16, 16)]
            mv = plsc.load_gather(mloc, [rv])
            ubuf[pl.ds(i * 16, 16)] = mv * wv
            return cc

        lax.fori_loop(0, CHUNK // 16, scale, jnp.int32(0))
        ds = [
            pltpu.async_copy(
                ubuf.at[pl.ds(g * EPG, EPG)], acc.at[cbuf.at[g]], sem, add=True
            )
            for g in range(GPC)
        ]
        for d in ds:
            d.wait()
        return carry

    lax.fori_loop(0, 200 // GPC, chunk, jnp.int32(0))
    plsc.subcore_barrier()

    @pl.when(c == 0)
    def _():
        pltpu.sync_copy(
            acc.at[pl.ds(s * RPS, RPS)], out_hbm.at[0, pl.ds(s * RPS, RPS)]
        )

    @pl.when(c == 1)
    def _():
        pltpu.sync_copy(
            acc.at[pl.ds(s * RPS, RPS)], out_hbm.at[1, pl.ds(s * RPS, RPS)]
        )


_sc_hop1 = functools.partial(
    pl.kernel,
    out_type=jax.ShapeDtypeStruct((2, NP), _f32),
    mesh=_MESH,
    compiler_params=pltpu.CompilerParams(needs_layout_passes=False, use_tc_tiling_on_sc=False),
    scratch_types=[
        pltpu.VMEM((CHUNK,), _i32),
        pltpu.VMEM((GPC, EPG), _i32),
        pltpu.VMEM((CHUNK,), _f32),
        pltpu.VMEM((CHUNK,), _f32),
        pltpu.VMEM((NP,), _f32),
        pltpu.VMEM_SHARED((NP,), _f32),
        pltpu.SemaphoreType.DMA,
    ],
)(_sc_hop1_body)


def _t0_body(degp_ref, x_ref, w_ref, dis_ref, dis2_ref, ma_ref, mb_ref, oa_ref):
    deg = degp_ref[0] + degp_ref[1]
    mask = deg > 0
    dis = jnp.where(mask, lax.rsqrt(deg), 0.0)
    dis2 = jnp.where(mask, 1.0 / deg, 0.0)
    dis_ref[...] = dis
    dis2_ref[...] = dis2
    x = x_ref[...]
    m0 = x * dis[:, None]
    ma_ref[...] = m0[:, :H]
    mb_ref[...] = m0[:, H:]
    oa_ref[...] = jnp.dot(x, w_ref[...], preferred_element_type=_f32)


def _t0(degp, x, w10):
    return pl.pallas_call(
        _t0_body,
        grid=(NB,),
        in_specs=[
            pl.BlockSpec((2, BN), lambda i: (0, i)),
            pl.BlockSpec((BN, F), lambda i: (i, 0)),
            pl.BlockSpec((F, F), lambda i: (0, 0)),
        ],
        out_specs=[
            pl.BlockSpec((BN,), lambda i: (i,)),
            pl.BlockSpec((BN,), lambda i: (i,)),
            pl.BlockSpec((BN, H), lambda i: (i, 0)),
            pl.BlockSpec((BN, H), lambda i: (i, 0)),
            pl.BlockSpec((BN, F), lambda i: (i, 0)),
        ],
        out_shape=[
            jax.ShapeDtypeStruct((NP,), _f32),
            jax.ShapeDtypeStruct((NP,), _f32),
            jax.ShapeDtypeStruct((NP, H), _f32),
            jax.ShapeDtypeStruct((NP, H), _f32),
            jax.ShapeDtypeStruct((NP, F), _f32),
        ],
    )(degp, x, w10)


def _thop_body(ta_ref, tb_ref, dis_ref, dis2_ref, oa_ref, w_ref, oao_ref,
               ma_ref, mb_ref):
    t = jnp.concatenate([ta_ref[...], tb_ref[...]], axis=1)
    dis = dis_ref[...]
    td = t * dis[:, None]
    oao_ref[...] = oa_ref[...] + jnp.dot(
        td, w_ref[...], preferred_element_type=_f32
    )
    m = t * dis2_ref[...][:, None]
    ma_ref[...] = m[:, :H]
    mb_ref[...] = m[:, H:]


def _thop(ta, tb, dis, dis2, oa, wk):
    return pl.pallas_call(
        _thop_body,
        grid=(NB,),
        in_specs=[
            pl.BlockSpec((BN, H), lambda i: (i, 0)),
            pl.BlockSpec((BN, H), lambda i: (i, 0)),
            pl.BlockSpec((BN,), lambda i: (i,)),
            pl.BlockSpec((BN,), lambda i: (i,)),
            pl.BlockSpec((BN, F), lambda i: (i, 0)),
            pl.BlockSpec((F, F), lambda i: (0, 0)),
        ],
        out_specs=[
            pl.BlockSpec((BN, F), lambda i: (i, 0)),
            pl.BlockSpec((BN, H), lambda i: (i, 0)),
            pl.BlockSpec((BN, H), lambda i: (i, 0)),
        ],
        out_shape=[
            jax.ShapeDtypeStruct((NP, F), _f32),
            jax.ShapeDtypeStruct((NP, H), _f32),
            jax.ShapeDtypeStruct((NP, H), _f32),
        ],
    )(ta, tb, dis, dis2, oa, wk)


def _tlend_body(ta_ref, tb_ref, dis_ref, oa_ref, w_ref, b_ref, wn_ref,
                oao_ref, ma_ref, mb_ref):
    t = jnp.concatenate([ta_ref[...], tb_ref[...]], axis=1)
    dis = dis_ref[...]
    td = t * dis[:, None]
    h = oa_ref[...] + jnp.dot(td, w_ref[...], preferred_element_type=_f32)
    h = jnp.maximum(h + b_ref[...][None, :], 0.0)
    oao_ref[...] = jnp.dot(h, wn_ref[...], preferred_element_type=_f32)
    m = h * dis[:, None]
    ma_ref[...] = m[:, :H]
    mb_ref[...] = m[:, H:]


def _tlend(ta, tb, dis, oa, wk, b, wn0):
    return pl.pallas_call(
        _tlend_body,
        grid=(NB,),
        in_specs=[
            pl.BlockSpec((BN, H), lambda i: (i, 0)),
            pl.BlockSpec((BN, H), lambda i: (i, 0)),
            pl.BlockSpec((BN,), lambda i: (i,)),
            pl.BlockSpec((BN, F), lambda i: (i, 0)),
            pl.BlockSpec((F, F), lambda i: (0, 0)),
            pl.BlockSpec((F,), lambda i: (0,)),
            pl.BlockSpec((F, F), lambda i: (0, 0)),
        ],
        out_specs=[
            pl.BlockSpec((BN, F), lambda i: (i, 0)),
            pl.BlockSpec((BN, H), lambda i: (i, 0)),
            pl.BlockSpec((BN, H), lambda i: (i, 0)),
        ],
        out_shape=[
            jax.ShapeDtypeStruct((NP, F), _f32),
            jax.ShapeDtypeStruct((NP, H), _f32),
            jax.ShapeDtypeStruct((NP, H), _f32),
        ],
    )(ta, tb, dis, oa, wk, b, wn0)


def _tl2end_body(ta_ref, tb_ref, dis_ref, oa_ref, w_ref, b_ref, w3_ref,
                 v_ref, m3_ref):
    t = jnp.concatenate([ta_ref[...], tb_ref[...]], axis=1)
    dis = dis_ref[...]
    td = t * dis[:, None]
    h = oa_ref[...] + jnp.dot(td, w_ref[...], preferred_element_type=_f32)
    h = jnp.maximum(h + b_ref[...][None, :], 0.0)
    v = jnp.dot(h, w3_ref[...], preferred_element_type=_f32)
    v_ref[...] = v
    m3_ref[...] = dis * v[:, 3]


def _tl2end(ta, tb, dis, oa, wk, b, w3c):
    return pl.pallas_call(
        _tl2end_body,
        grid=(NB,),
        in_specs=[
            pl.BlockSpec((BN, H), lambda i: (i, 0)),
            pl.BlockSpec((BN, H), lambda i: (i, 0)),
            pl.BlockSpec((BN,), lambda i: (i,)),
            pl.BlockSpec((BN, F), lambda i: (i, 0)),
            pl.BlockSpec((F, F), lambda i: (0, 0)),
            pl.BlockSpec((F,), lambda i: (0,)),
            pl.BlockSpec((F, 4), lambda i: (0, 0)),
        ],
        out_specs=[
            pl.BlockSpec((BN, 4), lambda i: (i, 0)),
            pl.BlockSpec((BN,), lambda i: (i,)),
        ],
        out_shape=[
            jax.ShapeDtypeStruct((NP, 4), _f32),
            jax.ShapeDtypeStruct((NP,), _f32),
        ],
    )(ta, tb, dis, oa, wk, b, w3c)


def _tw1_body(p_ref, v_ref, dis_ref, dis2_ref, m_ref, *, k):
    t = p_ref[0] + p_ref[1]
    m_ref[...] = dis_ref[...] * v_ref[:, k] + dis2_ref[...] * t


def _tw1(p, v, dis, dis2, k):
    return pl.pallas_call(
        functools.partial(_tw1_body, k=k),
        grid=(NB,),
        in_specs=[
            pl.BlockSpec((2, BN), lambda i: (0, i)),
            pl.BlockSpec((BN, 4), lambda i: (i, 0)),
            pl.BlockSpec((BN,), lambda i: (i,)),
            pl.BlockSpec((BN,), lambda i: (i,)),
        ],
        out_specs=pl.BlockSpec((BN,), lambda i: (i,)),
        out_shape=jax.ShapeDtypeStruct((NP,), _f32),
    )(p, v, dis, dis2)


def _tfinal_body(p_ref, v_ref, dis_ref, b3_ref, batch_ref, y_ref):
    i = pl.program_id(0)

    @pl.when(i == 0)
    def _():
        y_ref[...] = jnp.zeros_like(y_ref)

    out3 = v_ref[:, 0] + dis_ref[...] * (p_ref[0] + p_ref[1]) + b3_ref[0]
    b = batch_ref[0]
    onehot = (
        b[None, :] == lax.broadcasted_iota(jnp.int32, (G, 1), 0)
    ).astype(_f32)
    y_ref[...] += onehot @ out3[:, None]

    @pl.when(i == NB - 1)
    def _():
        y_ref[...] = jax.nn.sigmoid(y_ref[...])


def _tfinal(p, v, dis, b3, batch2d):
    return pl.pallas_call(
        _tfinal_body,
        grid=(NB,),
        in_specs=[
            pl.BlockSpec((2, BN), lambda i: (0, i)),
            pl.BlockSpec((BN, 4), lambda i: (i, 0)),
            pl.BlockSpec((BN,), lambda i: (i,)),
            pl.BlockSpec((1,), lambda i: (0,)),
            pl.BlockSpec((1, BN), lambda i: (0, i)),
        ],
        out_specs=pl.BlockSpec((G, 1), lambda i: (0, 0)),
        out_shape=jax.ShapeDtypeStruct((G, 1), _f32),
    )(p, v, dis, b3, batch2d)


def kernel(x, batch, edge_index, edge_weight, W1, b1, W2, b2, W3, b3):
    row = edge_index[0]
    col = edge_index[1]
    rowp = jnp.concatenate([row, jnp.zeros((EP - E,), _i32)])
    colp = jnp.concatenate([col, jnp.zeros((EP - E,), _i32)])
    wp = jnp.concatenate([edge_weight, jnp.zeros((EP - E,), _f32)])
    row2d = rowp.reshape(EP // EPG, EPG)
    col2d = colp.reshape(EP // EPG, EPG)
    x_pad = jnp.zeros((NP, F), _f32).at[:N].set(x)
    batch2d = jnp.full((NP,), -1, _i32).at[:N].set(batch).reshape(1, NP)
    z1 = jnp.zeros((RPS,), _f32)
    z2 = jnp.zeros((RPS, H), _f32)
    w3c = jnp.transpose(W3[:, :, 0])

    degp = _sc_deg(col2d, wp, z1)
    dis, dis2, ma, mb, oa = _t0(degp, x_pad, W1[0])

    for layer in range(2):
        Wl = W1 if layer == 0 else W2
        for k in (1, 2):
            ta, tb = _sc_hop32(ma, mb, row2d, col2d, wp, z2)
            oa, ma, mb = _thop(ta, tb, dis, dis2, oa, Wl[k])
        ta, tb = _sc_hop32(ma, mb, row2d, col2d, wp, z2)
        if layer == 0:
            oa, ma, mb = _tlend(ta, tb, dis, oa, W1[3], b1, W2[0])
        else:
            v, m = _tl2end(ta, tb, dis, oa, W2[3], b2, w3c)

    for k in (2, 1):
        p = _sc_hop1(m, rowp, col2d, wp, z1)
        m = _tw1(p, v, dis, dis2, k)
    p = _sc_hop1(m, rowp, col2d, wp, z1)
    return _tfinal(p, v, dis, b3, batch2d)


# speedup vs baseline: 7.6812x; 1.2064x over previous
import functools

import jax
import jax.numpy as jnp
from jax import lax
from jax.experimental import pallas as pl
from jax.experimental.pallas import tpu as pltpu
from jax.experimental.pallas import tpu_sc as plsc

N = 50000
E = 800000
G = 32
F = 64
H = 32

NP = 51200
EP = 819200
EPG = 128
GPC = 2
CHUNK = EPG * GPC
NSUB = 16
RPS = NP // NSUB
BN = 2048
NB = NP // BN

_MESH = plsc.VectorSubcoreMesh(core_axis_name="c", subcore_axis_name="s")
_f32 = jnp.float32
_i32 = jnp.int32


def _zero_slice(zsrc, acc, s):
    pltpu.sync_copy(zsrc, acc.at[pl.ds(s * RPS, RPS)])


def _sc_deg_body(col_hbm, w_hbm, z1_hbm, out_hbm, cbuf, wbuf, acc, sem):
    c = lax.axis_index("c")
    s = lax.axis_index("s")
    wid = c * NSUB + s
    _zero_slice(z1_hbm, acc, s)
    plsc.subcore_barrier()

    def chunk(ci, carry):
        gbase = wid * 200 + ci * GPC
        pltpu.sync_copy(col_hbm.at[pl.ds(gbase, GPC)], cbuf)
        pltpu.sync_copy(w_hbm.at[pl.ds(gbase * EPG, CHUNK)], wbuf)
        ds = [
            pltpu.async_copy(
                wbuf.at[pl.ds(g * EPG, EPG)], acc.at[cbuf.at[g]], sem, add=True
            )
            for g in range(GPC)
        ]
        for d in ds:
            d.wait()
        return carry

    lax.fori_loop(0, 200 // GPC, chunk, jnp.int32(0))
    plsc.subcore_barrier()

    @pl.when(c == 0)
    def _():
        pltpu.sync_copy(
            acc.at[pl.ds(s * RPS, RPS)], out_hbm.at[0, pl.ds(s * RPS, RPS)]
        )

    @pl.when(c == 1)
    def _():
        pltpu.sync_copy(
            acc.at[pl.ds(s * RPS, RPS)], out_hbm.at[1, pl.ds(s * RPS, RPS)]
        )


_sc_deg = functools.partial(
    pl.kernel,
    out_type=jax.ShapeDtypeStruct((2, NP), _f32),
    mesh=_MESH,
    compiler_params=pltpu.CompilerParams(needs_layout_passes=False, use_tc_tiling_on_sc=False),
    scratch_types=[
        pltpu.VMEM((GPC, EPG), _i32),
        pltpu.VMEM((CHUNK,), _f32),
        pltpu.VMEM_SHARED((NP,), _f32),
        pltpu.SemaphoreType.DMA,
    ],
)(_sc_deg_body)


def _hop32_gather(m_hbm, ebuf, gbuf, sem, gbase):
    return [
        pltpu.async_copy(
            m_hbm.at[ebuf.at[g, 0]], gbuf.at[pl.ds(g * EPG, EPG)], sem
        )
        for g in range(GPC)
    ]


def _hop32_drain_gather(m_hbm, ebuf, gbuf, sem):
    for g in range(GPC):
        pltpu.make_async_copy(
            m_hbm.at[ebuf.at[g, 0]], gbuf.at[pl.ds(g * EPG, EPG)], sem
        ).wait()


def _hop32_scale(ebuf, gbuf):
    for g in range(GPC):
        def blk(i, cc, g=g):
            for jj in range(16):
                eig = i * 16 + jj
                e = g * EPG + eig
                spi = plsc.load_gather(
                    ebuf,
                    [jnp.full((16,), g, _i32), jnp.full((16,), 2, _i32),
                     jnp.full((16,), eig, _i32)],
                )
                sp = plsc.bitcast(spi, _f32)
                gbuf[e, pl.ds(0, 16)] = gbuf[e, pl.ds(0, 16)] * sp
                gbuf[e, pl.ds(16, 16)] = gbuf[e, pl.ds(16, 16)] * sp
            return cc
        lax.fori_loop(0, EPG // 16, blk, jnp.int32(0))


def _hop32_scatter(acc, ebuf, gbuf, sem):
    return [
        pltpu.async_copy(
            gbuf.at[pl.ds(g * EPG, EPG)], acc.at[ebuf.at[g, 1]], sem, add=True
        )
        for g in range(GPC)
    ]


def _hop32_half(m_hbm, t_hbm, s, e_hbm, z2_hbm, ebuf0, ebuf1, gbuf0, gbuf1,
                acc, gsem0, gsem1, ssem0, ssem1):
    pltpu.sync_copy(z2_hbm, acc.at[pl.ds(s * RPS, RPS)])
    plsc.subcore_barrier()
    nch = 400 // GPC
    gps = s * 400

    pltpu.sync_copy(e_hbm.at[pl.ds(gps, GPC)], ebuf0)
    pltpu.sync_copy(e_hbm.at[pl.ds(gps + GPC, GPC)], ebuf1)
    _hop32_gather(m_hbm, ebuf0, gbuf0, gsem0, 0)
    _hop32_gather(m_hbm, ebuf1, gbuf1, gsem1, 0)

    def body(i, carry):
        p0 = jnp.minimum(2 * i + 2, nch - 1)
        p1 = jnp.minimum(2 * i + 3, nch - 1)
        _hop32_drain_gather(m_hbm, ebuf0, gbuf0, gsem0)
        _hop32_scale(ebuf0, gbuf0)
        s0 = _hop32_scatter(acc, ebuf0, gbuf0, ssem0)
        _hop32_drain_gather(m_hbm, ebuf1, gbuf1, gsem1)
        _hop32_scale(ebuf1, gbuf1)
        s1 = _hop32_scatter(acc, ebuf1, gbuf1, ssem1)
        for d in s0:
            d.wait()
        pltpu.sync_copy(e_hbm.at[pl.ds(gps + p0 * GPC, GPC)], ebuf0)
        _hop32_gather(m_hbm, ebuf0, gbuf0, gsem0, 0)
        for d in s1:
            d.wait()
        pltpu.sync_copy(e_hbm.at[pl.ds(gps + p1 * GPC, GPC)], ebuf1)
        _hop32_gather(m_hbm, ebuf1, gbuf1, gsem1, 0)
        return carry

    lax.fori_loop(0, nch // 2, body, jnp.int32(0))
    _hop32_drain_gather(m_hbm, ebuf0, gbuf0, gsem0)
    _hop32_drain_gather(m_hbm, ebuf1, gbuf1, gsem1)
    plsc.subcore_barrier()
    pltpu.sync_copy(acc.at[pl.ds(s * RPS, RPS)], t_hbm.at[pl.ds(s * RPS, RPS)])


def _sc_hop32_body(ma_hbm, mb_hbm, e_hbm, z2_hbm, ta_hbm, tb_hbm, ebuf0,
                   ebuf1, gbuf0, gbuf1, acc, gsem0, gsem1, ssem0, ssem1):
    c = lax.axis_index("c")
    s = lax.axis_index("s")

    @pl.when(c == 0)
    def _():
        _hop32_half(ma_hbm, ta_hbm, s, e_hbm, z2_hbm, ebuf0, ebuf1, gbuf0,
                    gbuf1, acc, gsem0, gsem1, ssem0, ssem1)

    @pl.when(c == 1)
    def _():
        _hop32_half(mb_hbm, tb_hbm, s, e_hbm, z2_hbm, ebuf0, ebuf1, gbuf0,
                    gbuf1, acc, gsem0, gsem1, ssem0, ssem1)


_sc_hop32 = functools.partial(
    pl.kernel,
    out_type=(
        jax.ShapeDtypeStruct((NP, H), _f32),
        jax.ShapeDtypeStruct((NP, H), _f32),
    ),
    mesh=_MESH,
    compiler_params=pltpu.CompilerParams(needs_layout_passes=False, use_tc_tiling_on_sc=False),
    scratch_types=[
        pltpu.VMEM((GPC, 3, EPG), _i32),
        pltpu.VMEM((GPC, 3, EPG), _i32),
        pltpu.VMEM((CHUNK, H), _f32),
        pltpu.VMEM((CHUNK, H), _f32),
        pltpu.VMEM_SHARED((NP, H), _f32),
        pltpu.SemaphoreType.DMA,
        pltpu.SemaphoreType.DMA,
        pltpu.SemaphoreType.DMA,
        pltpu.SemaphoreType.DMA,
    ],
)(_sc_hop32_body)


def _sc_hop1_body(m_hbm, row_hbm, col_hbm, w_hbm, z1_hbm, out_hbm, rbuf, cbuf,
                  wbuf, ubuf, mloc, acc, sem):
    c = lax.axis_index("c")
    s = lax.axis_index("s")
    wid = c * NSUB + s
    pltpu.sync_copy(m_hbm, mloc)
    _zero_slice(z1_hbm, acc, s)
    plsc.subcore_barrier()

    def chunk(ci, carry):
        gbase = wid * 200 + ci * GPC
        pltpu.sync_copy(row_hbm.at[pl.ds(gbase * EPG, CHUNK)], rbuf)
        pltpu.sync_copy(col_hbm.at[pl.ds(gbase, GPC)], cbuf)
        pltpu.sync_copy(w_hbm.at[pl.ds(gbase * EPG, CHUNK)], wbuf)

        def scale(i, cc):
            rv = rbuf[pl.ds(i * 16, 16)]
            wv = wbuf[pl.ds(i * 16, 16)]
            mv = plsc.load_gather(mloc, [rv])
            ubuf[pl.ds(i * 16, 16)] = mv * wv
            return cc

        lax.fori_loop(0, CHUNK // 16, scale, jnp.int32(0))
        ds = [
            pltpu.async_copy(
                ubuf.at[pl.ds(g * EPG, EPG)], acc.at[cbuf.at[g]], sem, add=True
            )
            for g in range(GPC)
        ]
        for d in ds:
            d.wait()
        return carry

    lax.fori_loop(0, 200 // GPC, chunk, jnp.int32(0))
    plsc.subcore_barrier()

    @pl.when(c == 0)
    def _():
        pltpu.sync_copy(
            acc.at[pl.ds(s * RPS, RPS)], out_hbm.at[0, pl.ds(s * RPS, RPS)]
        )

    @pl.when(c == 1)
    def _():
        pltpu.sync_copy(
            acc.at[pl.ds(s * RPS, RPS)], out_hbm.at[1, pl.ds(s * RPS, RPS)]
        )


_sc_hop1 = functools.partial(
    pl.kernel,
    out_type=jax.ShapeDtypeStruct((2, NP), _f32),
    mesh=_MESH,
    compiler_params=pltpu.CompilerParams(needs_layout_passes=False, use_tc_tiling_on_sc=False),
    scratch_types=[
        pltpu.VMEM((CHUNK,), _i32),
        pltpu.VMEM((GPC, EPG), _i32),
        pltpu.VMEM((CHUNK,), _f32),
        pltpu.VMEM((CHUNK,), _f32),
        pltpu.VMEM((NP,), _f32),
        pltpu.VMEM_SHARED((NP,), _f32),
        pltpu.SemaphoreType.DMA,
    ],
)(_sc_hop1_body)


def _t0_body(degp_ref, x_ref, w_ref, dis_ref, dis2_ref, ma_ref, mb_ref, oa_ref):
    deg = degp_ref[0] + degp_ref[1]
    mask = deg > 0
    dis = jnp.where(mask, lax.rsqrt(deg), 0.0)
    dis2 = jnp.where(mask, 1.0 / deg, 0.0)
    dis_ref[...] = dis
    dis2_ref[...] = dis2
    x = x_ref[...]
    m0 = x * dis[:, None]
    ma_ref[...] = m0[:, :H]
    mb_ref[...] = m0[:, H:]
    oa_ref[...] = jnp.dot(x, w_ref[...], preferred_element_type=_f32)


def _t0(degp, x, w10):
    return pl.pallas_call(
        _t0_body,
        grid=(NB,),
        in_specs=[
            pl.BlockSpec((2, BN), lambda i: (0, i)),
            pl.BlockSpec((BN, F), lambda i: (i, 0)),
            pl.BlockSpec((F, F), lambda i: (0, 0)),
        ],
        out_specs=[
            pl.BlockSpec((BN,), lambda i: (i,)),
            pl.BlockSpec((BN,), lambda i: (i,)),
            pl.BlockSpec((BN, H), lambda i: (i, 0)),
            pl.BlockSpec((BN, H), lambda i: (i, 0)),
            pl.BlockSpec((BN, F), lambda i: (i, 0)),
        ],
        out_shape=[
            jax.ShapeDtypeStruct((NP,), _f32),
            jax.ShapeDtypeStruct((NP,), _f32),
            jax.ShapeDtypeStruct((NP, H), _f32),
            jax.ShapeDtypeStruct((NP, H), _f32),
            jax.ShapeDtypeStruct((NP, F), _f32),
        ],
    )(degp, x, w10)


def _thop_body(ta_ref, tb_ref, dis_ref, dis2_ref, oa_ref, w_ref, oao_ref,
               ma_ref, mb_ref):
    t = jnp.concatenate([ta_ref[...], tb_ref[...]], axis=1)
    dis = dis_ref[...]
    td = t * dis[:, None]
    oao_ref[...] = oa_ref[...] + jnp.dot(
        td, w_ref[...], preferred_element_type=_f32
    )
    m = t * dis2_ref[...][:, None]
    ma_ref[...] = m[:, :H]
    mb_ref[...] = m[:, H:]


def _thop(ta, tb, dis, dis2, oa, wk):
    return pl.pallas_call(
        _thop_body,
        grid=(NB,),
        in_specs=[
            pl.BlockSpec((BN, H), lambda i: (i, 0)),
            pl.BlockSpec((BN, H), lambda i: (i, 0)),
            pl.BlockSpec((BN,), lambda i: (i,)),
            pl.BlockSpec((BN,), lambda i: (i,)),
            pl.BlockSpec((BN, F), lambda i: (i, 0)),
            pl.BlockSpec((F, F), lambda i: (0, 0)),
        ],
        out_specs=[
            pl.BlockSpec((BN, F), lambda i: (i, 0)),
            pl.BlockSpec((BN, H), lambda i: (i, 0)),
            pl.BlockSpec((BN, H), lambda i: (i, 0)),
        ],
        out_shape=[
            jax.ShapeDtypeStruct((NP, F), _f32),
            jax.ShapeDtypeStruct((NP, H), _f32),
            jax.ShapeDtypeStruct((NP, H), _f32),
        ],
    )(ta, tb, dis, dis2, oa, wk)


def _tlend_body(ta_ref, tb_ref, dis_ref, oa_ref, w_ref, b_ref, wn_ref,
                oao_ref, ma_ref, mb_ref):
    t = jnp.concatenate([ta_ref[...], tb_ref[...]], axis=1)
    dis = dis_ref[...]
    td = t * dis[:, None]
    h = oa_ref[...] + jnp.dot(td, w_ref[...], preferred_element_type=_f32)
    h = jnp.maximum(h + b_ref[...][None, :], 0.0)
    oao_ref[...] = jnp.dot(h, wn_ref[...], preferred_element_type=_f32)
    m = h * dis[:, None]
    ma_ref[...] = m[:, :H]
    mb_ref[...] = m[:, H:]


def _tlend(ta, tb, dis, oa, wk, b, wn0):
    return pl.pallas_call(
        _tlend_body,
        grid=(NB,),
        in_specs=[
            pl.BlockSpec((BN, H), lambda i: (i, 0)),
            pl.BlockSpec((BN, H), lambda i: (i, 0)),
            pl.BlockSpec((BN,), lambda i: (i,)),
            pl.BlockSpec((BN, F), lambda i: (i, 0)),
            pl.BlockSpec((F, F), lambda i: (0, 0)),
            pl.BlockSpec((F,), lambda i: (0,)),
            pl.BlockSpec((F, F), lambda i: (0, 0)),
        ],
        out_specs=[
            pl.BlockSpec((BN, F), lambda i: (i, 0)),
            pl.BlockSpec((BN, H), lambda i: (i, 0)),
            pl.BlockSpec((BN, H), lambda i: (i, 0)),
        ],
        out_shape=[
            jax.ShapeDtypeStruct((NP, F), _f32),
            jax.ShapeDtypeStruct((NP, H), _f32),
            jax.ShapeDtypeStruct((NP, H), _f32),
        ],
    )(ta, tb, dis, oa, wk, b, wn0)


def _tl2end_body(ta_ref, tb_ref, dis_ref, oa_ref, w_ref, b_ref, w3_ref,
                 v_ref, m3_ref):
    t = jnp.concatenate([ta_ref[...], tb_ref[...]], axis=1)
    dis = dis_ref[...]
    td = t * dis[:, None]
    h = oa_ref[...] + jnp.dot(td, w_ref[...], preferred_element_type=_f32)
    h = jnp.maximum(h + b_ref[...][None, :], 0.0)
    v = jnp.dot(h, w3_ref[...], preferred_element_type=_f32)
    v_ref[...] = v
    m3_ref[...] = dis * v[:, 3]


def _tl2end(ta, tb, dis, oa, wk, b, w3c):
    return pl.pallas_call(
        _tl2end_body,
        grid=(NB,),
        in_specs=[
            pl.BlockSpec((BN, H), lambda i: (i, 0)),
            pl.BlockSpec((BN, H), lambda i: (i, 0)),
            pl.BlockSpec((BN,), lambda i: (i,)),
            pl.BlockSpec((BN, F), lambda i: (i, 0)),
            pl.BlockSpec((F, F), lambda i: (0, 0)),
            pl.BlockSpec((F,), lambda i: (0,)),
            pl.BlockSpec((F, 4), lambda i: (0, 0)),
        ],
        out_specs=[
            pl.BlockSpec((BN, 4), lambda i: (i, 0)),
            pl.BlockSpec((BN,), lambda i: (i,)),
        ],
        out_shape=[
            jax.ShapeDtypeStruct((NP, 4), _f32),
            jax.ShapeDtypeStruct((NP,), _f32),
        ],
    )(ta, tb, dis, oa, wk, b, w3c)


def _tw1_body(p_ref, v_ref, dis_ref, dis2_ref, m_ref, *, k):
    t = p_ref[0] + p_ref[1]
    m_ref[...] = dis_ref[...] * v_ref[:, k] + dis2_ref[...] * t


def _tw1(p, v, dis, dis2, k):
    return pl.pallas_call(
        functools.partial(_tw1_body, k=k),
        grid=(NB,),
        in_specs=[
            pl.BlockSpec((2, BN), lambda i: (0, i)),
            pl.BlockSpec((BN, 4), lambda i: (i, 0)),
            pl.BlockSpec((BN,), lambda i: (i,)),
            pl.BlockSpec((BN,), lambda i: (i,)),
        ],
        out_specs=pl.BlockSpec((BN,), lambda i: (i,)),
        out_shape=jax.ShapeDtypeStruct((NP,), _f32),
    )(p, v, dis, dis2)


def _tfinal_body(p_ref, v_ref, dis_ref, b3_ref, batch_ref, y_ref):
    i = pl.program_id(0)

    @pl.when(i == 0)
    def _():
        y_ref[...] = jnp.zeros_like(y_ref)

    out3 = v_ref[:, 0] + dis_ref[...] * (p_ref[0] + p_ref[1]) + b3_ref[0]
    b = batch_ref[0]
    onehot = (
        b[None, :] == lax.broadcasted_iota(jnp.int32, (G, 1), 0)
    ).astype(_f32)
    y_ref[...] += onehot @ out3[:, None]

    @pl.when(i == NB - 1)
    def _():
        y_ref[...] = jax.nn.sigmoid(y_ref[...])


def _tfinal(p, v, dis, b3, batch2d):
    return pl.pallas_call(
        _tfinal_body,
        grid=(NB,),
        in_specs=[
            pl.BlockSpec((2, BN), lambda i: (0, i)),
            pl.BlockSpec((BN, 4), lambda i: (i, 0)),
            pl.BlockSpec((BN,), lambda i: (i,)),
            pl.BlockSpec((1,), lambda i: (0,)),
            pl.BlockSpec((1, BN), lambda i: (0, i)),
        ],
        out_specs=pl.BlockSpec((G, 1), lambda i: (0, 0)),
        out_shape=jax.ShapeDtypeStruct((G, 1), _f32),
    )(p, v, dis, b3, batch2d)


def kernel(x, batch, edge_index, edge_weight, W1, b1, W2, b2, W3, b3):
    row = edge_index[0]
    col = edge_index[1]
    rowp = jnp.concatenate([row, jnp.zeros((EP - E,), _i32)])
    colp = jnp.concatenate([col, jnp.zeros((EP - E,), _i32)])
    wp = jnp.concatenate([edge_weight, jnp.zeros((EP - E,), _f32)])
    col2d = colp.reshape(EP // EPG, EPG)
    epack = jnp.stack(
        [rowp.reshape(EP // EPG, EPG), col2d,
         wp.view(_i32).reshape(EP // EPG, EPG)], axis=1,
    )
    x_pad = jnp.zeros((NP, F), _f32).at[:N].set(x)
    batch2d = jnp.full((NP,), -1, _i32).at[:N].set(batch).reshape(1, NP)
    z1 = jnp.zeros((RPS,), _f32)
    z2 = jnp.zeros((RPS, H), _f32)
    w3c = jnp.transpose(W3[:, :, 0])

    degp = _sc_deg(col2d, wp, z1)
    dis, dis2, ma, mb, oa = _t0(degp, x_pad, W1[0])

    for layer in range(2):
        Wl = W1 if layer == 0 else W2
        for k in (1, 2):
            ta, tb = _sc_hop32(ma, mb, epack, z2)
            oa, ma, mb = _thop(ta, tb, dis, dis2, oa, Wl[k])
        ta, tb = _sc_hop32(ma, mb, epack, z2)
        if layer == 0:
            oa, ma, mb = _tlend(ta, tb, dis, oa, W1[3], b1, W2[0])
        else:
            v, m = _tl2end(ta, tb, dis, oa, W2[3], b2, w3c)

    for k in (2, 1):
        p = _sc_hop1(m, rowp, col2d, wp, z1)
        m = _tw1(p, v, dis, dis2, k)
    p = _sc_hop1(m, rowp, col2d, wp, z1)
    return _tfinal(p, v, dis, b3, batch2d)


# speedup vs baseline: 9.3571x; 1.2182x over previous
import functools

import jax
import jax.numpy as jnp
from jax import lax
from jax.experimental import pallas as pl
from jax.experimental.pallas import tpu as pltpu
from jax.experimental.pallas import tpu_sc as plsc

N = 50000
E = 800000
G = 32
F = 64
H = 32

NP = 51200
EP = 819200
EPG = 128
GPC = 2
CHUNK = EPG * GPC
NSUB = 16
RPS = NP // NSUB
BN = 2048
NB = NP // BN

_MESH = plsc.VectorSubcoreMesh(core_axis_name="c", subcore_axis_name="s")
_f32 = jnp.float32
_i32 = jnp.int32


def _zero_slice(zsrc, acc, s):
    pltpu.sync_copy(zsrc, acc.at[pl.ds(s * RPS, RPS)])


def _sc_deg_body(col_hbm, w_hbm, z1_hbm, out_hbm, cbuf, wbuf, acc, sem):
    c = lax.axis_index("c")
    s = lax.axis_index("s")
    wid = c * NSUB + s
    _zero_slice(z1_hbm, acc, s)
    plsc.subcore_barrier()

    def chunk(ci, carry):
        gbase = wid * 200 + ci * GPC
        pltpu.sync_copy(col_hbm.at[pl.ds(gbase, GPC)], cbuf)
        pltpu.sync_copy(w_hbm.at[pl.ds(gbase * EPG, CHUNK)], wbuf)
        ds = [
            pltpu.async_copy(
                wbuf.at[pl.ds(g * EPG, EPG)], acc.at[cbuf.at[g]], sem, add=True
            )
            for g in range(GPC)
        ]
        for d in ds:
            d.wait()
        return carry

    lax.fori_loop(0, 200 // GPC, chunk, jnp.int32(0))
    plsc.subcore_barrier()

    @pl.when(c == 0)
    def _():
        pltpu.sync_copy(
            acc.at[pl.ds(s * RPS, RPS)], out_hbm.at[0, pl.ds(s * RPS, RPS)]
        )

    @pl.when(c == 1)
    def _():
        pltpu.sync_copy(
            acc.at[pl.ds(s * RPS, RPS)], out_hbm.at[1, pl.ds(s * RPS, RPS)]
        )


_sc_deg = functools.partial(
    pl.kernel,
    out_type=jax.ShapeDtypeStruct((2, NP), _f32),
    mesh=_MESH,
    compiler_params=pltpu.CompilerParams(needs_layout_passes=False, use_tc_tiling_on_sc=False),
    scratch_types=[
        pltpu.VMEM((GPC, EPG), _i32),
        pltpu.VMEM((CHUNK,), _f32),
        pltpu.VMEM_SHARED((NP,), _f32),
        pltpu.SemaphoreType.DMA,
    ],
)(_sc_deg_body)


def _hop32_gather(m_hbm, ebuf, gbuf, sem, gbase):
    return [
        pltpu.async_copy(
            m_hbm.at[ebuf.at[g, 0]], gbuf.at[pl.ds(g * EPG, EPG)], sem
        )
        for g in range(GPC)
    ]


def _hop32_drain_gather(m_hbm, ebuf, gbuf, sem):
    for g in range(GPC):
        pltpu.make_async_copy(
            m_hbm.at[ebuf.at[g, 0]], gbuf.at[pl.ds(g * EPG, EPG)], sem
        ).wait()


def _hop32_scale(ebuf, gbuf):
    for g in range(GPC):
        def blk(i, cc, g=g):
            wv = plsc.bitcast(ebuf[g, 2, pl.ds(i * 16, 16)], _f32)
            for jj in range(16):
                e = g * EPG + i * 16 + jj
                sp = lax.gather(
                    wv, jnp.full((16, 1), jj, _i32),
                    lax.GatherDimensionNumbers(
                        offset_dims=(), collapsed_slice_dims=(0,),
                        start_index_map=(0,)),
                    (1,), mode=lax.GatherScatterMode.PROMISE_IN_BOUNDS)
                gbuf[e, pl.ds(0, 16)] = gbuf[e, pl.ds(0, 16)] * sp
                gbuf[e, pl.ds(16, 16)] = gbuf[e, pl.ds(16, 16)] * sp
            return cc
        lax.fori_loop(0, EPG // 16, blk, jnp.int32(0))


def _hop32_scatter(acc, ebuf, gbuf, sem):
    return [
        pltpu.async_copy(
            gbuf.at[pl.ds(g * EPG, EPG)], acc.at[ebuf.at[g, 1]], sem, add=True
        )
        for g in range(GPC)
    ]


def _hop32_half(m_hbm, t_hbm, s, e_hbm, z2_hbm, ebuf0, ebuf1, gbuf0, gbuf1,
                acc, gsem0, gsem1, ssem0, ssem1, isem0, isem1):
    pltpu.sync_copy(z2_hbm, acc.at[pl.ds(s * RPS, RPS)])
    plsc.subcore_barrier()
    nch = 400 // GPC
    gps = s * 400

    pltpu.sync_copy(e_hbm.at[pl.ds(gps, GPC)], ebuf0)
    pltpu.sync_copy(e_hbm.at[pl.ds(gps + GPC, GPC)], ebuf1)
    _hop32_gather(m_hbm, ebuf0, gbuf0, gsem0, 0)
    _hop32_gather(m_hbm, ebuf1, gbuf1, gsem1, 0)

    def body(i, carry):
        p0 = jnp.minimum(2 * i + 2, nch - 1)
        p1 = jnp.minimum(2 * i + 3, nch - 1)
        _hop32_drain_gather(m_hbm, ebuf0, gbuf0, gsem0)
        _hop32_scale(ebuf0, gbuf0)
        s0 = _hop32_scatter(acc, ebuf0, gbuf0, ssem0)
        _hop32_drain_gather(m_hbm, ebuf1, gbuf1, gsem1)
        _hop32_scale(ebuf1, gbuf1)
        s1 = _hop32_scatter(acc, ebuf1, gbuf1, ssem1)
        for d in s0:
            d.wait()
        i0 = pltpu.async_copy(e_hbm.at[pl.ds(gps + p0 * GPC, GPC)], ebuf0,
                              isem0)
        for d in s1:
            d.wait()
        i1 = pltpu.async_copy(e_hbm.at[pl.ds(gps + p1 * GPC, GPC)], ebuf1,
                              isem1)
        i0.wait()
        _hop32_gather(m_hbm, ebuf0, gbuf0, gsem0, 0)
        i1.wait()
        _hop32_gather(m_hbm, ebuf1, gbuf1, gsem1, 0)
        return carry

    lax.fori_loop(0, nch // 2, body, jnp.int32(0))
    _hop32_drain_gather(m_hbm, ebuf0, gbuf0, gsem0)
    _hop32_drain_gather(m_hbm, ebuf1, gbuf1, gsem1)
    plsc.subcore_barrier()
    pltpu.sync_copy(acc.at[pl.ds(s * RPS, RPS)], t_hbm.at[pl.ds(s * RPS, RPS)])


def _sc_hop32_body(ma_hbm, mb_hbm, e_hbm, z2_hbm, ta_hbm, tb_hbm, ebuf0,
                   ebuf1, gbuf0, gbuf1, acc, gsem0, gsem1, ssem0, ssem1,
                   isem0, isem1):
    c = lax.axis_index("c")
    s = lax.axis_index("s")

    @pl.when(c == 0)
    def _():
        _hop32_half(ma_hbm, ta_hbm, s, e_hbm, z2_hbm, ebuf0, ebuf1, gbuf0,
                    gbuf1, acc, gsem0, gsem1, ssem0, ssem1, isem0, isem1)

    @pl.when(c == 1)
    def _():
        _hop32_half(mb_hbm, tb_hbm, s, e_hbm, z2_hbm, ebuf0, ebuf1, gbuf0,
                    gbuf1, acc, gsem0, gsem1, ssem0, ssem1, isem0, isem1)


_sc_hop32 = functools.partial(
    pl.kernel,
    out_type=(
        jax.ShapeDtypeStruct((NP, H), _f32),
        jax.ShapeDtypeStruct((NP, H), _f32),
    ),
    mesh=_MESH,
    compiler_params=pltpu.CompilerParams(needs_layout_passes=False, use_tc_tiling_on_sc=False),
    scratch_types=[
        pltpu.VMEM((GPC, 3, EPG), _i32),
        pltpu.VMEM((GPC, 3, EPG), _i32),
        pltpu.VMEM((CHUNK, H), _f32),
        pltpu.VMEM((CHUNK, H), _f32),
        pltpu.VMEM_SHARED((NP, H), _f32),
        pltpu.SemaphoreType.DMA,
        pltpu.SemaphoreType.DMA,
        pltpu.SemaphoreType.DMA,
        pltpu.SemaphoreType.DMA,
        pltpu.SemaphoreType.DMA,
        pltpu.SemaphoreType.DMA,
    ],
)(_sc_hop32_body)


def _sc_hop1_body(m_hbm, row_hbm, col_hbm, w_hbm, z1_hbm, out_hbm, rbuf, cbuf,
                  wbuf, ubuf, mloc, acc, sem):
    c = lax.axis_index("c")
    s = lax.axis_index("s")
    wid = c * NSUB + s
    pltpu.sync_copy(m_hbm, mloc)
    _zero_slice(z1_hbm, acc, s)
    plsc.subcore_barrier()

    def chunk(ci, carry):
        gbase = wid * 200 + ci * GPC
        pltpu.sync_copy(row_hbm.at[pl.ds(gbase * EPG, CHUNK)], rbuf)
        pltpu.sync_copy(col_hbm.at[pl.ds(gbase, GPC)], cbuf)
        pltpu.sync_copy(w_hbm.at[pl.ds(gbase * EPG, CHUNK)], wbuf)

        def scale(i, cc):
            rv = rbuf[pl.ds(i * 16, 16)]
            wv = wbuf[pl.ds(i * 16, 16)]
            mv = plsc.load_gather(mloc, [rv])
            ubuf[pl.ds(i * 16, 16)] = mv * wv
            return cc

        lax.fori_loop(0, CHUNK // 16, scale, jnp.int32(0))
        ds = [
            pltpu.async_copy(
                ubuf.at[pl.ds(g * EPG, EPG)], acc.at[cbuf.at[g]], sem, add=True
            )
            for g in range(GPC)
        ]
        for d in ds:
            d.wait()
        return carry

    lax.fori_loop(0, 200 // GPC, chunk, jnp.int32(0))
    plsc.subcore_barrier()

    @pl.when(c == 0)
    def _():
        pltpu.sync_copy(
            acc.at[pl.ds(s * RPS, RPS)], out_hbm.at[0, pl.ds(s * RPS, RPS)]
        )

    @pl.when(c == 1)
    def _():
        pltpu.sync_copy(
            acc.at[pl.ds(s * RPS, RPS)], out_hbm.at[1, pl.ds(s * RPS, RPS)]
        )


_sc_hop1 = functools.partial(
    pl.kernel,
    out_type=jax.ShapeDtypeStruct((2, NP), _f32),
    mesh=_MESH,
    compiler_params=pltpu.CompilerParams(needs_layout_passes=False, use_tc_tiling_on_sc=False),
    scratch_types=[
        pltpu.VMEM((CHUNK,), _i32),
        pltpu.VMEM((GPC, EPG), _i32),
        pltpu.VMEM((CHUNK,), _f32),
        pltpu.VMEM((CHUNK,), _f32),
        pltpu.VMEM((NP,), _f32),
        pltpu.VMEM_SHARED((NP,), _f32),
        pltpu.SemaphoreType.DMA,
    ],
)(_sc_hop1_body)


def _t0_body(degp_ref, x_ref, w_ref, dis_ref, dis2_ref, ma_ref, mb_ref, oa_ref):
    deg = degp_ref[0] + degp_ref[1]
    mask = deg > 0
    dis = jnp.where(mask, lax.rsqrt(deg), 0.0)
    dis2 = jnp.where(mask, 1.0 / deg, 0.0)
    dis_ref[...] = dis
    dis2_ref[...] = dis2
    x = x_ref[...]
    m0 = x * dis[:, None]
    ma_ref[...] = m0[:, :H]
    mb_ref[...] = m0[:, H:]
    oa_ref[...] = jnp.dot(x, w_ref[...], preferred_element_type=_f32)


def _t0(degp, x, w10):
    return pl.pallas_call(
        _t0_body,
        grid=(NB,),
        in_specs=[
            pl.BlockSpec((2, BN), lambda i: (0, i)),
            pl.BlockSpec((BN, F), lambda i: (i, 0)),
            pl.BlockSpec((F, F), lambda i: (0, 0)),
        ],
        out_specs=[
            pl.BlockSpec((BN,), lambda i: (i,)),
            pl.BlockSpec((BN,), lambda i: (i,)),
            pl.BlockSpec((BN, H), lambda i: (i, 0)),
            pl.BlockSpec((BN, H), lambda i: (i, 0)),
            pl.BlockSpec((BN, F), lambda i: (i, 0)),
        ],
        out_shape=[
            jax.ShapeDtypeStruct((NP,), _f32),
            jax.ShapeDtypeStruct((NP,), _f32),
            jax.ShapeDtypeStruct((NP, H), _f32),
            jax.ShapeDtypeStruct((NP, H), _f32),
            jax.ShapeDtypeStruct((NP, F), _f32),
        ],
    )(degp, x, w10)


def _thop_body(ta_ref, tb_ref, dis_ref, dis2_ref, oa_ref, w_ref, oao_ref,
               ma_ref, mb_ref):
    t = jnp.concatenate([ta_ref[...], tb_ref[...]], axis=1)
    dis = dis_ref[...]
    td = t * dis[:, None]
    oao_ref[...] = oa_ref[...] + jnp.dot(
        td, w_ref[...], preferred_element_type=_f32
    )
    m = t * dis2_ref[...][:, None]
    ma_ref[...] = m[:, :H]
    mb_ref[...] = m[:, H:]


def _thop(ta, tb, dis, dis2, oa, wk):
    return pl.pallas_call(
        _thop_body,
        grid=(NB,),
        in_specs=[
            pl.BlockSpec((BN, H), lambda i: (i, 0)),
            pl.BlockSpec((BN, H), lambda i: (i, 0)),
            pl.BlockSpec((BN,), lambda i: (i,)),
            pl.BlockSpec((BN,), lambda i: (i,)),
            pl.BlockSpec((BN, F), lambda i: (i, 0)),
            pl.BlockSpec((F, F), lambda i: (0, 0)),
        ],
        out_specs=[
            pl.BlockSpec((BN, F), lambda i: (i, 0)),
            pl.BlockSpec((BN, H), lambda i: (i, 0)),
            pl.BlockSpec((BN, H), lambda i: (i, 0)),
        ],
        out_shape=[
            jax.ShapeDtypeStruct((NP, F), _f32),
            jax.ShapeDtypeStruct((NP, H), _f32),
            jax.ShapeDtypeStruct((NP, H), _f32),
        ],
    )(ta, tb, dis, dis2, oa, wk)


def _tlend_body(ta_ref, tb_ref, dis_ref, oa_ref, w_ref, b_ref, wn_ref,
                oao_ref, ma_ref, mb_ref):
    t = jnp.concatenate([ta_ref[...], tb_ref[...]], axis=1)
    dis = dis_ref[...]
    td = t * dis[:, None]
    h = oa_ref[...] + jnp.dot(td, w_ref[...], preferred_element_type=_f32)
    h = jnp.maximum(h + b_ref[...][None, :], 0.0)
    oao_ref[...] = jnp.dot(h, wn_ref[...], preferred_element_type=_f32)
    m = h * dis[:, None]
    ma_ref[...] = m[:, :H]
    mb_ref[...] = m[:, H:]


def _tlend(ta, tb, dis, oa, wk, b, wn0):
    return pl.pallas_call(
        _tlend_body,
        grid=(NB,),
        in_specs=[
            pl.BlockSpec((BN, H), lambda i: (i, 0)),
            pl.BlockSpec((BN, H), lambda i: (i, 0)),
            pl.BlockSpec((BN,), lambda i: (i,)),
            pl.BlockSpec((BN, F), lambda i: (i, 0)),
            pl.BlockSpec((F, F), lambda i: (0, 0)),
            pl.BlockSpec((F,), lambda i: (0,)),
            pl.BlockSpec((F, F), lambda i: (0, 0)),
        ],
        out_specs=[
            pl.BlockSpec((BN, F), lambda i: (i, 0)),
            pl.BlockSpec((BN, H), lambda i: (i, 0)),
            pl.BlockSpec((BN, H), lambda i: (i, 0)),
        ],
        out_shape=[
            jax.ShapeDtypeStruct((NP, F), _f32),
            jax.ShapeDtypeStruct((NP, H), _f32),
            jax.ShapeDtypeStruct((NP, H), _f32),
        ],
    )(ta, tb, dis, oa, wk, b, wn0)


def _tl2end_body(ta_ref, tb_ref, dis_ref, oa_ref, w_ref, b_ref, w3_ref,
                 v_ref, m3_ref):
    t = jnp.concatenate([ta_ref[...], tb_ref[...]], axis=1)
    dis = dis_ref[...]
    td = t * dis[:, None]
    h = oa_ref[...] + jnp.dot(td, w_ref[...], preferred_element_type=_f32)
    h = jnp.maximum(h + b_ref[...][None, :], 0.0)
    v = jnp.dot(h, w3_ref[...], preferred_element_type=_f32)
    v_ref[...] = v
    m3_ref[...] = dis * v[:, 3]


def _tl2end(ta, tb, dis, oa, wk, b, w3c):
    return pl.pallas_call(
        _tl2end_body,
        grid=(NB,),
        in_specs=[
            pl.BlockSpec((BN, H), lambda i: (i, 0)),
            pl.BlockSpec((BN, H), lambda i: (i, 0)),
            pl.BlockSpec((BN,), lambda i: (i,)),
            pl.BlockSpec((BN, F), lambda i: (i, 0)),
            pl.BlockSpec((F, F), lambda i: (0, 0)),
            pl.BlockSpec((F,), lambda i: (0,)),
            pl.BlockSpec((F, 4), lambda i: (0, 0)),
        ],
        out_specs=[
            pl.BlockSpec((BN, 4), lambda i: (i, 0)),
            pl.BlockSpec((BN,), lambda i: (i,)),
        ],
        out_shape=[
            jax.ShapeDtypeStruct((NP, 4), _f32),
            jax.ShapeDtypeStruct((NP,), _f32),
        ],
    )(ta, tb, dis, oa, wk, b, w3c)


def _tw1_body(p_ref, v_ref, dis_ref, dis2_ref, m_ref, *, k):
    t = p_ref[0] + p_ref[1]
    m_ref[...] = dis_ref[...] * v_ref[:, k] + dis2_ref[...] * t


def _tw1(p, v, dis, dis2, k):
    return pl.pallas_call(
        functools.partial(_tw1_body, k=k),
        grid=(NB,),
        in_specs=[
            pl.BlockSpec((2, BN), lambda i: (0, i)),
            pl.BlockSpec((BN, 4), lambda i: (i, 0)),
            pl.BlockSpec((BN,), lambda i: (i,)),
            pl.BlockSpec((BN,), lambda i: (i,)),
        ],
        out_specs=pl.BlockSpec((BN,), lambda i: (i,)),
        out_shape=jax.ShapeDtypeStruct((NP,), _f32),
    )(p, v, dis, dis2)


def _tfinal_body(p_ref, v_ref, dis_ref, b3_ref, batch_ref, y_ref):
    i = pl.program_id(0)

    @pl.when(i == 0)
    def _():
        y_ref[...] = jnp.zeros_like(y_ref)

    out3 = v_ref[:, 0] + dis_ref[...] * (p_ref[0] + p_ref[1]) + b3_ref[0]
    b = batch_ref[0]
    onehot = (
        b[None, :] == lax.broadcasted_iota(jnp.int32, (G, 1), 0)
    ).astype(_f32)
    y_ref[...] += onehot @ out3[:, None]

    @pl.when(i == NB - 1)
    def _():
        y_ref[...] = jax.nn.sigmoid(y_ref[...])


def _tfinal(p, v, dis, b3, batch2d):
    return pl.pallas_call(
        _tfinal_body,
        grid=(NB,),
        in_specs=[
            pl.BlockSpec((2, BN), lambda i: (0, i)),
            pl.BlockSpec((BN, 4), lambda i: (i, 0)),
            pl.BlockSpec((BN,), lambda i: (i,)),
            pl.BlockSpec((1,), lambda i: (0,)),
            pl.BlockSpec((1, BN), lambda i: (0, i)),
        ],
        out_specs=pl.BlockSpec((G, 1), lambda i: (0, 0)),
        out_shape=jax.ShapeDtypeStruct((G, 1), _f32),
    )(p, v, dis, b3, batch2d)


def kernel(x, batch, edge_index, edge_weight, W1, b1, W2, b2, W3, b3):
    row = edge_index[0]
    col = edge_index[1]
    rowp = jnp.concatenate([row, jnp.zeros((EP - E,), _i32)])
    colp = jnp.concatenate([col, jnp.zeros((EP - E,), _i32)])
    wp = jnp.concatenate([edge_weight, jnp.zeros((EP - E,), _f32)])
    col2d = colp.reshape(EP // EPG, EPG)
    epack = jnp.stack(
        [rowp.reshape(EP // EPG, EPG), col2d,
         wp.view(_i32).reshape(EP // EPG, EPG)], axis=1,
    )
    x_pad = jnp.zeros((NP, F), _f32).at[:N].set(x)
    batch2d = jnp.full((NP,), -1, _i32).at[:N].set(batch).reshape(1, NP)
    z1 = jnp.zeros((RPS,), _f32)
    z2 = jnp.zeros((RPS, H), _f32)
    w3c = jnp.transpose(W3[:, :, 0])

    degp = _sc_deg(col2d, wp, z1)
    dis, dis2, ma, mb, oa = _t0(degp, x_pad, W1[0])

    for layer in range(2):
        Wl = W1 if layer == 0 else W2
        for k in (1, 2):
            ta, tb = _sc_hop32(ma, mb, epack, z2)
            oa, ma, mb = _thop(ta, tb, dis, dis2, oa, Wl[k])
        ta, tb = _sc_hop32(ma, mb, epack, z2)
        if layer == 0:
            oa, ma, mb = _tlend(ta, tb, dis, oa, W1[3], b1, W2[0])
        else:
            v, m = _tl2end(ta, tb, dis, oa, W2[3], b2, w3c)

    for k in (2, 1):
        p = _sc_hop1(m, rowp, col2d, wp, z1)
        m = _tw1(p, v, dis, dis2, k)
    p = _sc_hop1(m, rowp, col2d, wp, z1)
    return _tfinal(p, v, dis, b3, batch2d)
